# baseline (speedup 1.0000x reference)
import functools

import jax
import jax.numpy as jnp
from jax import lax
from jax.experimental import pallas as pl
from jax.experimental.pallas import tpu as pltpu
from jax.experimental.pallas import tpu_sc as plsc

B = 16384
EMB = 64
LANES = 16
USERS = 1000000

_info = plsc.get_sparse_core_info()
NC = _info.num_cores
NS = _info.num_subcores
NW = NC * NS
BPW = B // NW
CHUNK = 128
NCHUNK = BPW // CHUNK

TR_CH = 4096
SPLIT = 97 * TR_CH


def _tr_body(src_ref, dst_ref):
    ident = jnp.eye(EMB, dtype=jnp.float32)
    dst_ref[...] = lax.dot_general(
        src_ref[...], ident, (((0,), (0,)), ((), ())),
        preferred_element_type=jnp.float32)


def _detile(table_t, n):
    return pl.pallas_call(
        _tr_body,
        grid=(pl.cdiv(n, TR_CH),),
        in_specs=[pl.BlockSpec((EMB, TR_CH), lambda j: (0, j))],
        out_specs=pl.BlockSpec((TR_CH, EMB), lambda j: (j, 0)),
        out_shape=jax.ShapeDtypeStruct((n, EMB), jnp.float32),
    )(table_t)


@functools.partial(
    pl.kernel,
    out_type=jax.ShapeDtypeStruct((B,), jnp.float32),
    mesh=plsc.VectorSubcoreMesh(core_axis_name="c", subcore_axis_name="s"),
    compiler_params=pltpu.CompilerParams(use_tc_tiling_on_sc=False),
    scratch_types=[
        pltpu.VMEM((NCHUNK, CHUNK), jnp.int32),
        pltpu.VMEM((NCHUNK, CHUNK), jnp.int32),
        pltpu.VMEM((NCHUNK, CHUNK), jnp.int32),
        pltpu.VMEM((NCHUNK, CHUNK), jnp.int32),
        pltpu.VMEM((BPW, EMB), jnp.float32),
        pltpu.VMEM((BPW, EMB), jnp.float32),
        pltpu.VMEM((BPW, EMB), jnp.float32),
        pltpu.VMEM((BPW,), jnp.float32),
        pltpu.SemaphoreType.DMA,
    ],
)
def _dot_kernel(uid_hbm, mid_hbm, utlo_hbm, uthi_hbm, mt_hbm, out_hbm,
                uorig_v, ulo_v, uhi_v, midx_v, ulrows_v, uhrows_v, mrows_v,
                out_v, sem):
    wid = lax.axis_index("s") * NC + lax.axis_index("c")

    pltpu.sync_copy(uid_hbm.at[wid], uorig_v)
    pltpu.sync_copy(mid_hbm.at[wid], midx_v)
    for g in range(NCHUNK):
        for v in range(CHUNK // LANES):
            idx = uorig_v[g, pl.ds(v * LANES, LANES)]
            uhi_v[g, pl.ds(v * LANES, LANES)] = jnp.maximum(
                idx - SPLIT, 0)
            ulo_v[g, pl.ds(v * LANES, LANES)] = jnp.minimum(
                idx, SPLIT - 1)

    copies = []
    for g in range(NCHUNK):
        copies.append(pltpu.async_copy(
            utlo_hbm.at[ulo_v.at[g]],
            ulrows_v.at[pl.ds(g * CHUNK, CHUNK)], sem))
        copies.append(pltpu.async_copy(
            uthi_hbm.at[uhi_v.at[g]],
            uhrows_v.at[pl.ds(g * CHUNK, CHUNK)], sem))
        copies.append(pltpu.async_copy(
            mt_hbm.at[midx_v.at[g]],
            mrows_v.at[pl.ds(g * CHUNK, CHUNK)], sem))
    for c in copies:
        c.wait()

    lane = lax.iota(jnp.int32, 16)
    perms = [(lane + s) & (LANES - 1) for s in (8, 4, 2, 1)]
    gd = lax.GatherDimensionNumbers(
        offset_dims=(), collapsed_slice_dims=(0,), start_index_map=(0,))

    def shuffle(x, p):
        return lax.gather(x, p[:, None], gd, slice_sizes=(1,),
                          mode=lax.GatherScatterMode.PROMISE_IN_BOUNDS)

    def group_body(g, carry):
        base_r = g * LANES
        iorig = uorig_v[g // (CHUNK // LANES),
                        pl.ds((g % (CHUNK // LANES)) * LANES, LANES)]
        acc = jnp.zeros((LANES,), jnp.float32)
        for i in range(LANES):
            r = base_r + i
            hi_i = iorig[i]
            t = jnp.zeros((LANES,), jnp.float32)
            for k in range(EMB // 16):
                ul = ulrows_v[r, pl.ds(k * 16, 16)]
                uh = uhrows_v[r, pl.ds(k * 16, 16)]
                u = jnp.where(hi_i >= SPLIT, uh, ul)
                t = t + u * mrows_v[r, pl.ds(k * 16, 16)]
            for p in perms:
                t = t + shuffle(t, p)
            acc = jnp.where(lane == i, t, acc)
        out_v[pl.ds(base_r, LANES)] = acc
        return carry

    lax.fori_loop(0, BPW // LANES, group_body, 0)

    pltpu.sync_copy(out_v, out_hbm.at[pl.ds(wid * BPW, BPW)])


def kernel(userId, movieId, user_table, movie_table):
    uid = userId.reshape(NW, NCHUNK, CHUNK)
    mid = movieId.reshape(NW, NCHUNK, CHUNK)
    ut_lo = _detile(user_table.T, SPLIT)
    ut_hi = user_table[SPLIT:]
    mt = _detile(movie_table.T, movie_table.shape[0])
    out = _dot_kernel(uid, mid, ut_lo, ut_hi, mt)
    return out.reshape(B, 1)


# speedup vs baseline: 1.0426x; 1.0426x over previous
import functools

import jax
import jax.numpy as jnp
from jax import lax
from jax.experimental import pallas as pl
from jax.experimental.pallas import tpu as pltpu
from jax.experimental.pallas import tpu_sc as plsc

B = 16384
EMB = 64
LANES = 16
MOVIES = 100000

_info = plsc.get_sparse_core_info()
NC = _info.num_cores
NS = _info.num_subcores
NW = NC * NS
BPW = B // NW
CHUNK = 128
NCHUNK = BPW // CHUNK
NPASS = 2
CPP = NCHUNK // NPASS

U_CH = 4096
SPLIT = 97 * U_CH
M_CH = 4096


def _tr_body(src_ref, dst_ref):
    h = src_ref.shape[1] // 2
    ident = jnp.eye(EMB, dtype=jnp.float32)
    dst_ref[:, 0:EMB] = lax.dot_general(
        src_ref[:, 0:h], ident, (((0,), (0,)), ((), ())),
        preferred_element_type=jnp.float32)
    dst_ref[:, EMB:2 * EMB] = lax.dot_general(
        src_ref[:, h:2 * h], ident, (((0,), (0,)), ((), ())),
        preferred_element_type=jnp.float32)


def _detile2(table_t, n, ch):
    nblk = pl.cdiv(n, ch)
    return pl.pallas_call(
        _tr_body,
        grid=(nblk,),
        in_specs=[pl.BlockSpec((EMB, ch), lambda j: (0, j))],
        out_specs=pl.BlockSpec((ch // 2, 2 * EMB), lambda j: (j, 0)),
        out_shape=jax.ShapeDtypeStruct((nblk * ch // 2, 2 * EMB),
                                       jnp.float32),
    )(table_t)


@functools.partial(
    pl.kernel,
    out_type=jax.ShapeDtypeStruct((B,), jnp.float32),
    mesh=plsc.VectorSubcoreMesh(core_axis_name="c", subcore_axis_name="s"),
    compiler_params=pltpu.CompilerParams(use_tc_tiling_on_sc=False),
    scratch_types=[
        pltpu.VMEM((NCHUNK, CHUNK), jnp.int32),
        pltpu.VMEM((NCHUNK, CHUNK), jnp.int32),
        pltpu.VMEM((NCHUNK, CHUNK), jnp.int32),
        pltpu.VMEM((NCHUNK, CHUNK), jnp.int32),
        pltpu.VMEM((NCHUNK, CHUNK), jnp.int32),
        pltpu.VMEM((NCHUNK, CHUNK), jnp.int32),
        pltpu.VMEM((BPW // NPASS, 2 * EMB), jnp.float32),
        pltpu.VMEM((BPW // NPASS, EMB), jnp.float32),
        pltpu.VMEM((BPW // NPASS, 2 * EMB), jnp.float32),
        pltpu.VMEM((BPW,), jnp.float32),
        pltpu.SemaphoreType.DMA,
    ],
)
def _dot_kernel(uid_hbm, mid_hbm, utlo_hbm, uthi_hbm, mt_hbm, out_hbm,
                uorig_v, ulo_v, uhi_v, midx_v, offu_v, offm_v,
                ulrows_v, uhrows_v, mrows_v, out_v, sem):
    wid = lax.axis_index("s") * NC + lax.axis_index("c")

    pltpu.sync_copy(uid_hbm.at[wid], uorig_v)
    pltpu.sync_copy(mid_hbm.at[wid], midx_v)

    for g in range(NCHUNK):
        for v in range(CHUNK // LANES):
            sl = pl.ds(v * LANES, LANES)
            uidx = uorig_v[g, sl]
            uhi_v[g, sl] = jnp.maximum(uidx - SPLIT, 0)
            uc = jnp.minimum(uidx, SPLIT - 1)
            ulo_v[g, sl] = ((uc >> 12) << 11) | (uc & 2047)
            offu_v[g, sl] = ((uc >> 11) & 1) * EMB
            midx = midx_v[g, sl]
            offm_v[g, sl] = ((midx >> 11) & 1) * EMB
            midx_v[g, sl] = ((midx >> 12) << 11) | (midx & 2047)

    lane = lax.iota(jnp.int32, 16)
    perms = [(lane + s) & (LANES - 1) for s in (8, 4, 2, 1)]
    gd = lax.GatherDimensionNumbers(
        offset_dims=(), collapsed_slice_dims=(0,), start_index_map=(0,))

    def shuffle(x, p):
        return lax.gather(x, p[:, None], gd, slice_sizes=(1,),
                          mode=lax.GatherScatterMode.PROMISE_IN_BOUNDS)

    for p_i in range(NPASS):
        copies = []
        for gg in range(CPP):
            g = p_i * CPP + gg
            copies.append(pltpu.async_copy(
                utlo_hbm.at[ulo_v.at[g]],
                ulrows_v.at[pl.ds(gg * CHUNK, CHUNK)], sem))
            copies.append(pltpu.async_copy(
                uthi_hbm.at[uhi_v.at[g]],
                uhrows_v.at[pl.ds(gg * CHUNK, CHUNK)], sem))
            copies.append(pltpu.async_copy(
                mt_hbm.at[midx_v.at[g]],
                mrows_v.at[pl.ds(gg * CHUNK, CHUNK)], sem))
        for c in copies:
            c.wait()

        def group_body(g2, carry):
            base_r = g2 * LANES
            gch = p_i * CPP + g2 // (CHUNK // LANES)
            gsl = pl.ds((g2 % (CHUNK // LANES)) * LANES, LANES)
            iorig = uorig_v[gch, gsl]
            ioffu = offu_v[gch, gsl]
            ioffm = offm_v[gch, gsl]
            acc = jnp.zeros((LANES,), jnp.float32)
            for i in range(LANES):
                r = base_r + i
                hi_i = iorig[i]
                ou = ioffu[i]
                om = ioffm[i]
                t = jnp.zeros((LANES,), jnp.float32)
                for k in range(EMB // 16):
                    ul = ulrows_v[r, pl.ds(ou + k * 16, 16)]
                    uh = uhrows_v[r, pl.ds(k * 16, 16)]
                    u = jnp.where(hi_i >= SPLIT, uh, ul)
                    t = t + u * mrows_v[r, pl.ds(om + k * 16, 16)]
                for p in perms:
                    t = t + shuffle(t, p)
                acc = jnp.where(lane == i, t, acc)
            out_v[pl.ds(p_i * (BPW // NPASS) + base_r, LANES)] = acc
            return carry

        lax.fori_loop(0, BPW // NPASS // LANES, group_body, 0)

    pltpu.sync_copy(out_v, out_hbm.at[pl.ds(wid * BPW, BPW)])


def kernel(userId, movieId, user_table, movie_table):
    uid = userId.reshape(NW, NCHUNK, CHUNK)
    mid = movieId.reshape(NW, NCHUNK, CHUNK)
    ut_lo = _detile2(user_table.T, SPLIT, U_CH)
    ut_hi = user_table[SPLIT:]
    mt = _detile2(movie_table.T, MOVIES, M_CH)
    out = _dot_kernel(uid, mid, ut_lo, ut_hi, mt)
    return out.reshape(B, 1)


# speedup vs baseline: 1.0471x; 1.0043x over previous
import functools

import jax
import jax.numpy as jnp
from jax import lax
from jax.experimental import pallas as pl
from jax.experimental.pallas import tpu as pltpu
from jax.experimental.pallas import tpu_sc as plsc

B = 16384
EMB = 64
LANES = 16
MOVIES = 100000

_info = plsc.get_sparse_core_info()
NC = _info.num_cores
NS = _info.num_subcores
NW = NC * NS
BPW = B // NW
CHUNK = 128
NCHUNK = BPW // CHUNK
NPASS = 2
CPP = NCHUNK // NPASS

U_CH = 16384
SPLIT = 24 * U_CH
HB = 13
HBIT = 1 << HB


def _tr_body(src_ref, dst_ref):
    h = src_ref.shape[1] // 2
    ident = jnp.eye(EMB, dtype=jnp.float32)
    dst_ref[:, 0:EMB] = lax.dot_general(
        src_ref[:, 0:h], ident, (((0,), (0,)), ((), ())),
        preferred_element_type=jnp.float32)
    dst_ref[:, EMB:2 * EMB] = lax.dot_general(
        src_ref[:, h:2 * h], ident, (((0,), (0,)), ((), ())),
        preferred_element_type=jnp.float32)


def _detile2(table_t, n, ch):
    nblk = n // ch
    return pl.pallas_call(
        _tr_body,
        grid=(nblk,),
        in_specs=[pl.BlockSpec((EMB, ch), lambda j: (0, j))],
        out_specs=pl.BlockSpec((ch // 2, 2 * EMB), lambda j: (j, 0)),
        out_shape=jax.ShapeDtypeStruct((nblk * ch // 2, 2 * EMB),
                                       jnp.float32),
    )(table_t)


@functools.partial(
    pl.kernel,
    out_type=jax.ShapeDtypeStruct((B,), jnp.float32),
    mesh=plsc.VectorSubcoreMesh(core_axis_name="c", subcore_axis_name="s"),
    compiler_params=pltpu.CompilerParams(use_tc_tiling_on_sc=False),
    scratch_types=[
        pltpu.VMEM((NCHUNK, CHUNK), jnp.int32),
        pltpu.VMEM((NCHUNK, CHUNK), jnp.int32),
        pltpu.VMEM((NCHUNK, CHUNK), jnp.int32),
        pltpu.VMEM((NCHUNK, CHUNK), jnp.int32),
        pltpu.VMEM((BPW // NPASS, 2 * EMB), jnp.float32),
        pltpu.VMEM((BPW // NPASS, EMB), jnp.float32),
        pltpu.VMEM((BPW // NPASS, EMB), jnp.float32),
        pltpu.VMEM((BPW,), jnp.float32),
        pltpu.SemaphoreType.DMA,
    ],
)
def _dot_kernel(uid_hbm, mid_hbm, utlo_hbm, uthi_hbm, mt_hbm, out_hbm,
                uorig_v, ulo_v, uhi_v, midx_v,
                ulrows_v, uhrows_v, mrows_v, out_v, sem):
    wid = lax.axis_index("s") * NC + lax.axis_index("c")

    pltpu.sync_copy(uid_hbm.at[wid], uorig_v)
    pltpu.sync_copy(mid_hbm.at[wid], midx_v)

    for g in range(NCHUNK):
        for v in range(CHUNK // LANES):
            sl = pl.ds(v * LANES, LANES)
            uidx = uorig_v[g, sl]
            uhi_v[g, sl] = jnp.maximum(uidx - SPLIT, 0)
            uc = jnp.minimum(uidx, SPLIT - 1)
            ulo_v[g, sl] = ((uc >> (HB + 1)) << HB) | (uc & (HBIT - 1))

    lane = lax.iota(jnp.int32, 16)
    perms = [(lane + s) & (LANES - 1) for s in (8, 4, 2, 1)]
    gd = lax.GatherDimensionNumbers(
        offset_dims=(), collapsed_slice_dims=(0,), start_index_map=(0,))

    def shuffle(x, p):
        return lax.gather(x, p[:, None], gd, slice_sizes=(1,),
                          mode=lax.GatherScatterMode.PROMISE_IN_BOUNDS)

    for p_i in range(NPASS):
        copies = []
        for gg in range(CPP):
            g = p_i * CPP + gg
            copies.append(pltpu.async_copy(
                utlo_hbm.at[ulo_v.at[g]],
                ulrows_v.at[pl.ds(gg * CHUNK, CHUNK)], sem))
            copies.append(pltpu.async_copy(
                uthi_hbm.at[uhi_v.at[g]],
                uhrows_v.at[pl.ds(gg * CHUNK, CHUNK)], sem))
            copies.append(pltpu.async_copy(
                mt_hbm.at[midx_v.at[g]],
                mrows_v.at[pl.ds(gg * CHUNK, CHUNK)], sem))
        for c in copies:
            c.wait()

        def group_body(g2, carry):
            base_r = g2 * LANES
            gch = p_i * CPP + g2 // (CHUNK // LANES)
            gsl = pl.ds((g2 % (CHUNK // LANES)) * LANES, LANES)
            iorig = uorig_v[gch, gsl]
            acc0 = jnp.zeros((LANES,), jnp.float32)
            acc1 = jnp.zeros((LANES,), jnp.float32)
            acch = jnp.zeros((LANES,), jnp.float32)
            for i in range(LANES):
                r = base_r + i
                t0 = jnp.zeros((LANES,), jnp.float32)
                t1 = jnp.zeros((LANES,), jnp.float32)
                th = jnp.zeros((LANES,), jnp.float32)
                for k in range(EMB // 16):
                    m = mrows_v[r, pl.ds(k * 16, 16)]
                    t0 = t0 + m * ulrows_v[r, pl.ds(k * 16, 16)]
                    t1 = t1 + m * ulrows_v[r, pl.ds(EMB + k * 16, 16)]
                    th = th + m * uhrows_v[r, pl.ds(k * 16, 16)]
                for p in perms:
                    t0 = t0 + shuffle(t0, p)
                    t1 = t1 + shuffle(t1, p)
                    th = th + shuffle(th, p)
                sel = lane == i
                acc0 = jnp.where(sel, t0, acc0)
                acc1 = jnp.where(sel, t1, acc1)
                acch = jnp.where(sel, th, acch)
            ishalf1 = (iorig >> HB) & 1
            accl = jnp.where(ishalf1 == 1, acc1, acc0)
            acc = jnp.where(iorig >= SPLIT, acch, accl)
            out_v[pl.ds(p_i * (BPW // NPASS) + base_r, LANES)] = acc
            return carry

        lax.fori_loop(0, BPW // NPASS // LANES, group_body, 0)

    pltpu.sync_copy(out_v, out_hbm.at[pl.ds(wid * BPW, BPW)])


def kernel(userId, movieId, user_table, movie_table):
    uid = userId.reshape(NW, NCHUNK, CHUNK)
    mid = movieId.reshape(NW, NCHUNK, CHUNK)
    ut_lo = _detile2(user_table.T, SPLIT, U_CH)
    ut_hi = user_table[SPLIT:]
    out = _dot_kernel(uid, mid, ut_lo, ut_hi, movie_table)
    return out.reshape(B, 1)


# speedup vs baseline: 1.4531x; 1.3878x over previous
import functools

import jax
import jax.numpy as jnp
from jax import lax
from jax.experimental import pallas as pl
from jax.experimental.pallas import tpu as pltpu
from jax.experimental.pallas import tpu_sc as plsc

B = 16384
EMB = 64
LANES = 16

_info = plsc.get_sparse_core_info()
NC = _info.num_cores
NS = _info.num_subcores
NW = NC * NS
BPW = B // NW
CHUNK = 128
NCHUNK = BPW // CHUNK
NPASS = 2
CPP = NCHUNK // NPASS
RPP = BPW // NPASS
GPP = RPP // LANES


@functools.partial(
    pl.kernel,
    out_type=jax.ShapeDtypeStruct((B,), jnp.float32),
    mesh=plsc.VectorSubcoreMesh(core_axis_name="c", subcore_axis_name="s"),
    compiler_params=pltpu.CompilerParams(use_tc_tiling_on_sc=True),
    scratch_types=[
        pltpu.VMEM((NCHUNK, CHUNK), jnp.int32),
        pltpu.VMEM((NCHUNK, CHUNK), jnp.int32),
        pltpu.VMEM((NCHUNK, CHUNK), jnp.int32),
        pltpu.VMEM((NCHUNK, CHUNK), jnp.int32),
        pltpu.VMEM((RPP, 2 * EMB), jnp.float32),
        pltpu.VMEM((RPP, 2 * EMB), jnp.float32),
        pltpu.VMEM((RPP,), jnp.float32),
        pltpu.VMEM((RPP,), jnp.float32),
        pltpu.VMEM((RPP,), jnp.float32),
        pltpu.VMEM((RPP,), jnp.float32),
        pltpu.VMEM((BPW,), jnp.float32),
        pltpu.SemaphoreType.DMA,
    ],
)
def _dot_kernel(uid_hbm, mid_hbm, ut_hbm, mt_hbm, out_hbm,
                uorig_v, morig_v, upk_v, mpk_v,
                urows_v, mrows_v, t00_v, t01_v, t10_v, t11_v,
                out_v, sem):
    wid = lax.axis_index("s") * NC + lax.axis_index("c")

    pltpu.sync_copy(uid_hbm.at[wid], uorig_v)
    pltpu.sync_copy(mid_hbm.at[wid], morig_v)

    for g in range(NCHUNK):
        for v in range(CHUNK // LANES):
            sl = pl.ds(v * LANES, LANES)
            upk_v[g, sl] = uorig_v[g, sl] >> 1
            mpk_v[g, sl] = morig_v[g, sl] >> 1

    lane = lax.iota(jnp.int32, 16)
    perms = [(lane + s) & (LANES - 1) for s in (8, 4, 2, 1)]
    gd = lax.GatherDimensionNumbers(
        offset_dims=(), collapsed_slice_dims=(0,), start_index_map=(0,))

    def shuffle(x, p):
        return lax.gather(x, p[:, None], gd, slice_sizes=(1,),
                          mode=lax.GatherScatterMode.PROMISE_IN_BOUNDS)

    for p_i in range(NPASS):
        copies = []
        for gg in range(CPP):
            g = p_i * CPP + gg
            copies.append(pltpu.async_copy(
                ut_hbm.at[upk_v.at[g]],
                urows_v.at[pl.ds(gg * CHUNK, CHUNK)], sem))
            copies.append(pltpu.async_copy(
                mt_hbm.at[mpk_v.at[g]],
                mrows_v.at[pl.ds(gg * CHUNK, CHUNK)], sem))
        for c in copies:
            c.wait()

        for tv, a, b in ((t00_v, 0, 0), (t01_v, 0, 1),
                         (t10_v, 1, 0), (t11_v, 1, 1)):
            def variant_body(g2, carry, tv=tv, a=a, b=b):
                base_r = g2 * LANES
                acc = jnp.zeros((LANES,), jnp.float32)
                for i in range(LANES):
                    r = base_r + i
                    t = jnp.zeros((LANES,), jnp.float32)
                    for k in range(EMB // 16):
                        t = t + (urows_v[r, pl.ds(a * EMB + k * 16, 16)]
                                 * mrows_v[r, pl.ds(b * EMB + k * 16, 16)])
                    for p in perms:
                        t = t + shuffle(t, p)
                    acc = jnp.where(lane == i, t, acc)
                tv[pl.ds(base_r, LANES)] = acc
                return carry
            lax.fori_loop(0, GPP, variant_body, 0)

        def select_body(g2, carry):
            base_r = g2 * LANES
            gch = p_i * CPP + g2 // (CHUNK // LANES)
            gsl = pl.ds((g2 % (CHUNK // LANES)) * LANES, LANES)
            ua = uorig_v[gch, gsl] & 1
            mb = morig_v[gch, gsl] & 1
            sl = pl.ds(base_r, LANES)
            t0 = jnp.where(mb == 1, t01_v[sl], t00_v[sl])
            t1 = jnp.where(mb == 1, t11_v[sl], t10_v[sl])
            out_v[pl.ds(p_i * RPP + base_r, LANES)] = jnp.where(
                ua == 1, t1, t0)
            return carry
        lax.fori_loop(0, GPP, select_body, 0)

    pltpu.sync_copy(out_v, out_hbm.at[pl.ds(wid * BPW, BPW)])


def kernel(userId, movieId, user_table, movie_table):
    uid = userId.reshape(NW, NCHUNK, CHUNK)
    mid = movieId.reshape(NW, NCHUNK, CHUNK)
    ut2 = user_table.reshape(user_table.shape[0] // 2, 2 * EMB)
    mt2 = movie_table.reshape(movie_table.shape[0] // 2, 2 * EMB)
    out = _dot_kernel(uid, mid, ut2, mt2)
    return out.reshape(B, 1)
